# Initial kernel scaffold; baseline (speedup 1.0000x reference)
import jax
import jax.numpy as jnp
from jax.experimental import pallas as pl


def kernel(x, hyperedge_index, W_conv, b_conv, W_mlp, b_mlp, gamma, beta):
    raise NotImplementedError("write your pallas kernel here")


# speedup vs baseline: 17.0309x; 17.0309x over previous
import functools

import jax
import jax.numpy as jnp
from jax import lax
from jax.experimental import pallas as pl
from jax.experimental.pallas import tpu as pltpu
from jax.experimental.pallas import tpu_sc as plsc

N_NODES = 10000
N_EDGES = 10000
N_INC = 320000
D = 128
H = 64
W = 80
NC = 2
NS = 16
K = 80
NB = N_INC // (NS * K)
ROWS_PER_TILE = N_NODES // NS
ECH = 125


def _sc_pass_body(table_hbm, gidx_hbm, sidx_hbm, zrows_hbm, out_hbm,
                  acc_shared, gbuf, sbuf, rows_a, rows_b, ebuf, sem_a, sem_b):
    c = lax.axis_index("c")
    s = lax.axis_index("s")

    pltpu.sync_copy(zrows_hbm, ebuf)
    for ch in range(ROWS_PER_TILE // ECH):
        pltpu.sync_copy(ebuf, acc_shared.at[pl.ds(s * ROWS_PER_TILE + ch * ECH, ECH)])
    plsc.subcore_barrier()

    pltpu.sync_copy(gidx_hbm.at[c, s], gbuf)
    pltpu.sync_copy(sidx_hbm.at[s], sbuf)

    cp0 = pltpu.async_copy(table_hbm.at[gbuf.at[0]], rows_a, sem_a)

    def step(j, _):
        even = lax.rem(j, 2) == 0

        @pl.when(even)
        def _():
            @pl.when(j + 1 < NB)
            def _():
                pltpu.async_copy(table_hbm.at[gbuf.at[j + 1]], rows_b, sem_b)
            pltpu.make_async_copy(table_hbm.at[gbuf.at[0]], rows_a, sem_a).wait()
            pltpu.sync_copy(rows_a, acc_shared.at[sbuf.at[j]], add=True)

        @pl.when(jnp.logical_not(even))
        def _():
            @pl.when(j + 1 < NB)
            def _():
                pltpu.async_copy(table_hbm.at[gbuf.at[j + 1]], rows_a, sem_a)
            pltpu.make_async_copy(table_hbm.at[gbuf.at[0]], rows_b, sem_b).wait()
            pltpu.sync_copy(rows_b, acc_shared.at[sbuf.at[j]], add=True)

        return 0

    lax.fori_loop(0, NB, step, 0)
    del cp0
    plsc.subcore_barrier()

    ones_first = jnp.where(lax.iota(jnp.int32, 16) == 0,
                           jnp.float32(1.0), jnp.float32(0.0))
    for ch in range(ROWS_PER_TILE // ECH):
        base = s * ROWS_PER_TILE + ch * ECH
        pltpu.sync_copy(acc_shared.at[pl.ds(base, ECH)], ebuf)

        def erow(i, _):
            cnt = ebuf[i, pl.ds(H, 16)][0]
            cntv = jnp.full((16,), cnt, jnp.float32)
            invv = jnp.where(cntv > 0.0, 1.0 / cntv, jnp.float32(0.0))
            for q in range(H // 16):
                ebuf[i, pl.ds(q * 16, 16)] = ebuf[i, pl.ds(q * 16, 16)] * invv
            ebuf[i, pl.ds(H, 16)] = ones_first
            return 0

        lax.fori_loop(0, ECH, erow, 0)
        pltpu.sync_copy(ebuf, out_hbm.at[c, pl.ds(base, ECH)])


@functools.partial(jax.jit, static_argnames=())
def _sc_pass(table, gidx, sidx, zrows):
    mesh = plsc.VectorSubcoreMesh(core_axis_name="c", subcore_axis_name="s",
                                  num_cores=NC, num_subcores=NS)
    return pl.kernel(
        _sc_pass_body,
        out_type=jax.ShapeDtypeStruct((NC, N_NODES, W), jnp.float32),
        mesh=mesh,
        scratch_types=[
            pltpu.VMEM_SHARED((N_NODES, W), jnp.float32),
            pltpu.VMEM((NB, K), jnp.int32),
            pltpu.VMEM((NB, K), jnp.int32),
            pltpu.VMEM((K, W), jnp.float32),
            pltpu.VMEM((K, W), jnp.float32),
            pltpu.VMEM((ECH, W), jnp.float32),
            pltpu.SemaphoreType.DMA,
            pltpu.SemaphoreType.DMA,
        ],
        compiler_params=pltpu.CompilerParams(use_tc_tiling_on_sc=False),
    )(table, gidx, sidx, zrows)


def _matmul_body(x_ref, w_ref, o_ref):
    o_ref[...] = jnp.dot(x_ref[...], w_ref[...],
                         preferred_element_type=jnp.float32)


def _mlp_body(conv_ref, x_ref, bc_ref, wm_ref, bm_ref, g_ref, b_ref, o_ref):
    h = conv_ref[...] + bc_ref[...]
    h = jnp.dot(h, wm_ref[...], preferred_element_type=jnp.float32)
    h = h + bm_ref[...]
    mean = jnp.mean(h, axis=0, keepdims=True)
    var = jnp.mean((h - mean) ** 2, axis=0, keepdims=True)
    h = (h - mean) * lax.rsqrt(var + 1e-5)
    h = g_ref[...] * h + b_ref[...]
    h = jnp.where(h >= 0, h, 0.01 * h)
    r = h + x_ref[...]
    o_ref[...] = jnp.where(r >= 0, r, 0.01 * r)


def kernel(x, hyperedge_index, W_conv, b_conv, W_mlp, b_mlp, gamma, beta):
    node_idx = hyperedge_index[0].astype(jnp.int32)
    edge_idx = hyperedge_index[1].astype(jnp.int32)

    xw = pl.pallas_call(
        _matmul_body,
        out_shape=jax.ShapeDtypeStruct((N_NODES, D), jnp.float32),
    )(x, W_conv)

    aux = jnp.concatenate(
        [jnp.ones((N_NODES, 1), jnp.float32),
         jnp.zeros((N_NODES, W - H - 1), jnp.float32)], axis=1)
    table1 = jnp.concatenate(
        [jnp.concatenate([xw[:, :H], aux], axis=1),
         jnp.concatenate([xw[:, H:], aux], axis=1)], axis=0)

    nidx = node_idx.reshape(NS, NB, K)
    eidx = edge_idx.reshape(NS, NB, K)
    nadj = jnp.stack([nidx, nidx + N_NODES])
    eadj = jnp.stack([eidx, eidx + N_EDGES])
    zrows = jnp.zeros((ECH, W), jnp.float32)

    s1 = _sc_pass(table1, nadj, eidx, zrows)
    s2 = _sc_pass(s1.reshape(NC * N_NODES, W), eadj, nidx, zrows)
    conv = jnp.concatenate([s2[0, :, :H], s2[1, :, :H]], axis=1)

    return pl.pallas_call(
        _mlp_body,
        out_shape=jax.ShapeDtypeStruct((N_NODES, D), jnp.float32),
    )(conv, x, b_conv.reshape(1, D), W_mlp, b_mlp.reshape(1, D),
      gamma.reshape(1, D), beta.reshape(1, D))


# speedup vs baseline: 20.1754x; 1.1846x over previous
import functools

import jax
import jax.numpy as jnp
from jax import lax
from jax.experimental import pallas as pl
from jax.experimental.pallas import tpu as pltpu
from jax.experimental.pallas import tpu_sc as plsc

N_NODES = 10000
N_EDGES = 10000
N_INC = 320000
D = 128
H = 64
W = 80
NC = 2
NS = 16
K = 128
INC_PER_TILE = N_INC // NS
NB = -(-INC_PER_TILE // K)
PAD = NB * K - INC_PER_TILE
TRASH = N_NODES
ACC_ROWS = N_NODES + 8
RING = 4
ROWS_PER_TILE = N_NODES // NS
ECH = 125


def _sc_pass_body(table_hbm, idx_hbm, zrows_hbm, out_hbm,
                  acc_shared, iring, rows, ebuf, isem, gsem, ssem):
    c = lax.axis_index("c")
    s = lax.axis_index("s")

    pltpu.sync_copy(zrows_hbm, ebuf)
    for ch in range(ROWS_PER_TILE // ECH):
        pltpu.sync_copy(ebuf, acc_shared.at[pl.ds(s * ROWS_PER_TILE + ch * ECH, ECH)])
    plsc.subcore_barrier()

    def fire_idx(j):
        pltpu.async_copy(idx_hbm.at[c, s, j], iring.at[lax.rem(j, RING)],
                         isem.at[lax.rem(j, RING)])

    def wait_idx(j):
        pltpu.make_async_copy(idx_hbm.at[c, s, j],
                              iring.at[lax.rem(j, RING)],
                              isem.at[lax.rem(j, RING)]).wait()

    def fire_gather(j):
        slot = lax.rem(j, RING)
        pltpu.async_copy(table_hbm.at[iring.at[slot, 0]], rows.at[slot],
                         gsem.at[slot])

    def wait_gather(j):
        slot = lax.rem(j, RING)
        pltpu.make_async_copy(table_hbm.at[iring.at[slot, 0]],
                              rows.at[slot], gsem.at[slot]).wait()

    def fire_scatter(j):
        slot = lax.rem(j, RING)
        pltpu.async_copy(rows.at[slot], acc_shared.at[iring.at[slot, 1]],
                         ssem.at[slot], add=True)

    def wait_scatter(j):
        slot = lax.rem(j, RING)
        pltpu.make_async_copy(rows.at[slot],
                              acc_shared.at[iring.at[slot, 1]],
                              ssem.at[slot]).wait()

    for t in range(RING):
        fire_idx(t)
    for g in range(2):
        wait_idx(g)
        fire_gather(g)

    def step(j, _):
        @pl.when(jnp.logical_and(j >= 1, j - 1 + RING < NB))
        def _():
            wait_scatter(j - 1)
            fire_idx(j - 1 + RING)

        @pl.when(j + 2 < NB)
        def _():
            wait_idx(j + 2)
            fire_gather(j + 2)

        wait_gather(j)
        fire_scatter(j)
        return 0

    lax.fori_loop(0, NB, step, 0)

    def drain(r, _):
        wait_scatter(r)
        return 0

    lax.fori_loop(NB - RING, NB, drain, 0)
    plsc.subcore_barrier()

    ones_first = jnp.where(lax.iota(jnp.int32, 16) == 0,
                           jnp.float32(1.0), jnp.float32(0.0))
    for ch in range(ROWS_PER_TILE // ECH):
        base = s * ROWS_PER_TILE + ch * ECH
        pltpu.sync_copy(acc_shared.at[pl.ds(base, ECH)], ebuf)

        def erow(i, _):
            cnt = ebuf[i, pl.ds(H, 16)][0]
            cntv = jnp.full((16,), cnt, jnp.float32)
            invv = jnp.where(cntv > 0.0, 1.0 / cntv, jnp.float32(0.0))
            for q in range(H // 16):
                ebuf[i, pl.ds(q * 16, 16)] = ebuf[i, pl.ds(q * 16, 16)] * invv
            ebuf[i, pl.ds(H, 16)] = ones_first
            return 0

        lax.fori_loop(0, ECH, erow, 0)
        pltpu.sync_copy(ebuf, out_hbm.at[c, pl.ds(base, ECH)])


@functools.partial(jax.jit, static_argnames=())
def _sc_pass(table, idx, zrows):
    mesh = plsc.VectorSubcoreMesh(core_axis_name="c", subcore_axis_name="s",
                                  num_cores=NC, num_subcores=NS)
    return pl.kernel(
        _sc_pass_body,
        out_type=jax.ShapeDtypeStruct((NC, N_NODES, W), jnp.float32),
        mesh=mesh,
        scratch_types=[
            pltpu.VMEM_SHARED((ACC_ROWS, W), jnp.float32),
            pltpu.VMEM((RING, 2, K), jnp.int32),
            pltpu.VMEM((RING, K, W), jnp.float32),
            pltpu.VMEM((ECH, W), jnp.float32),
            pltpu.SemaphoreType.DMA((RING,)),
            pltpu.SemaphoreType.DMA((RING,)),
            pltpu.SemaphoreType.DMA((RING,)),
        ],
        compiler_params=pltpu.CompilerParams(use_tc_tiling_on_sc=False),
    )(table, idx, zrows)


def _table_body(x_ref, w_ref, o_ref):
    xwh = jnp.dot(x_ref[...], w_ref[0], preferred_element_type=jnp.float32)
    col = lax.broadcasted_iota(jnp.int32, (N_NODES, W - H), 1)
    aux = jnp.where(col == 0, jnp.float32(1.0), jnp.float32(0.0))
    o_ref[...] = jnp.concatenate([xwh, aux], axis=1)


def _mlp_body(s2_ref, x_ref, bc_ref, wm_ref, bm_ref, g_ref, b_ref, o_ref):
    h = (jnp.dot(s2_ref[0, :, :H], wm_ref[:H, :],
                 preferred_element_type=jnp.float32)
         + jnp.dot(s2_ref[1, :, :H], wm_ref[H:, :],
                   preferred_element_type=jnp.float32)
         + jnp.dot(bc_ref[...], wm_ref[...],
                   preferred_element_type=jnp.float32))
    h = h + bm_ref[...]
    mean = jnp.mean(h, axis=0, keepdims=True)
    var = jnp.mean((h - mean) ** 2, axis=0, keepdims=True)
    h = (h - mean) * lax.rsqrt(var + 1e-5)
    h = g_ref[...] * h + b_ref[...]
    h = jnp.where(h >= 0, h, 0.01 * h)
    r = h + x_ref[...]
    o_ref[...] = jnp.where(r >= 0, r, 0.01 * r)


def kernel(x, hyperedge_index, W_conv, b_conv, W_mlp, b_mlp, gamma, beta):
    node_idx = hyperedge_index[0].astype(jnp.int32)
    edge_idx = hyperedge_index[1].astype(jnp.int32)

    table1 = pl.pallas_call(
        _table_body,
        grid=(NC,),
        in_specs=[
            pl.BlockSpec((N_NODES, D), lambda c: (0, 0)),
            pl.BlockSpec((1, D, H), lambda c: (c, 0, 0)),
        ],
        out_specs=pl.BlockSpec((N_NODES, W), lambda c: (c, 0)),
        out_shape=jax.ShapeDtypeStruct((NC * N_NODES, W), jnp.float32),
    )(x, jnp.stack([W_conv[:, :H], W_conv[:, H:]]))

    n2 = node_idx.reshape(NS, INC_PER_TILE)
    e2 = edge_idx.reshape(NS, INC_PER_TILE)
    padg = jnp.zeros((NS, PAD), jnp.int32)
    pads = jnp.full((NS, PAD), TRASH, jnp.int32)
    n_g = jnp.concatenate([n2, padg], axis=1).reshape(NS, NB, K)
    n_s = jnp.concatenate([n2, pads], axis=1).reshape(NS, NB, K)
    e_g = jnp.concatenate([e2, padg], axis=1).reshape(NS, NB, K)
    e_s = jnp.concatenate([e2, pads], axis=1).reshape(NS, NB, K)
    nadj = jnp.stack([n_g, n_g + N_NODES])
    eadj = jnp.stack([e_g, e_g + N_EDGES])
    comb1 = jnp.stack(
        [nadj, jnp.broadcast_to(e_s, (NC, NS, NB, K))], axis=3)
    comb2 = jnp.stack(
        [eadj, jnp.broadcast_to(n_s, (NC, NS, NB, K))], axis=3)
    zrows = jnp.zeros((ECH, W), jnp.float32)

    s1 = _sc_pass(table1, comb1, zrows)
    s2 = _sc_pass(s1.reshape(NC * N_NODES, W), comb2, zrows)

    return pl.pallas_call(
        _mlp_body,
        out_shape=jax.ShapeDtypeStruct((N_NODES, D), jnp.float32),
    )(s2, x, b_conv.reshape(1, D), W_mlp, b_mlp.reshape(1, D),
      gamma.reshape(1, D), beta.reshape(1, D))
